# baseline (speedup 1.0000x reference)
import jax
import jax.numpy as jnp
from jax import lax
from jax.experimental import pallas as pl
from jax.experimental.pallas import tpu as pltpu
from jax.experimental.pallas import tpu_sc as plsc

N = 10000
NP = 10240
E = 160000
NT = 16
EPT = E // NT
NPT = NP // NT
L = 16


def _leaky(e):
    return jnp.maximum(e, 0.0) + 0.2 * jnp.minimum(e, 0.0)


def _body(src_h, dst_h, x_h, c_h,
          out_h,
          srcv, dstv, tb0, tb1, tb2, tb3,
          a0, a1, a2, a3, o0, o1, o2, rng, obuf, zbuf, idxv, pv, pbuf, cv,
          sh_a0, sh_a1, sh_a2, sh_a3, sh_tbl, sh_part, sem):
    tid = lax.axis_index("s")
    ebase = tid * EPT
    nbase = tid * NPT
    zeros = jnp.zeros((L,), jnp.float32)

    def bc(k):
        return plsc.load_gather(cv, [jnp.full((L,), k, jnp.int32)])

    pltpu.sync_copy(src_h.at[pl.ds(ebase, EPT)], srcv)
    pltpu.sync_copy(dst_h.at[pl.ds(ebase, EPT)], dstv)
    pltpu.sync_copy(x_h, tb2)
    pltpu.sync_copy(c_h, cv)

    idc = lax.iota(jnp.int32, L)

    @pl.loop(0, NPT // L)
    def _(j):
        zbuf[pl.ds(j * L, L)] = zeros

    @pl.loop(0, NP // L)
    def _(j):
        idxv[pl.ds(j * L, L)] = j * L + idc

    sh_accs = [sh_a0, sh_a1, sh_a2, sh_a3]

    def zero_accs(refs):
        @pl.loop(0, NP // L)
        def _(j):
            for r in refs:
                r[pl.ds(j * L, L)] = zeros
        for k in range(len(refs)):
            pltpu.sync_copy(zbuf, sh_accs[k].at[pl.ds(nbase, NPT)])

    def merge_accs(refs):
        plsc.subcore_barrier()
        for k, r in enumerate(refs):
            pltpu.sync_copy(r, sh_accs[k].at[idxv], add=True)
        plsc.subcore_barrier()
        for k in range(len(refs)):
            pltpu.sync_copy(sh_accs[k].at[pl.ds(nbase, NPT)], rng.at[k])

    cs1 = bc(0)
    cd1 = bc(1)
    zero_accs([a0, a1])

    @pl.loop(0, EPT // L, unroll=4)
    def _(i):
        s16 = srcv[pl.ds(i * L, L)]
        d16 = dstv[pl.ds(i * L, L)]
        xs = plsc.load_gather(tb2, [s16])
        xd = plsc.load_gather(tb2, [d16])
        w = jnp.exp(_leaky(cs1 * xs + cd1 * xd))
        plsc.addupdate_scatter(a0, [d16], w)
        plsc.addupdate_scatter(a1, [d16], w * xs)

    merge_accs([a0, a1])

    w10, w11, w12 = bc(2), bc(3), bc(4)
    b10, b11, b12 = bc(5), bc(6), bc(7)

    @pl.loop(0, NPT // L)
    def _(j):
        sl16 = pl.ds(j * L, L)
        xv = tb2[pl.ds(nbase + j * L, L)]
        wv = jnp.exp(_leaky(cs1 * xv + cd1 * xv))
        den = rng[0, sl16] + wv
        s = rng[1, sl16] + wv * xv
        r = s / den
        o0[sl16] = jnp.maximum(w10 * r + b10, 0.0)
        o1[sl16] = jnp.maximum(w11 * r + b11, 0.0)
        o2[sl16] = jnp.maximum(w12 * r + b12, 0.0)

    w2 = [[bc(8 + 3 * i + j) for j in range(3)] for i in range(3)]
    as2 = [bc(17 + j) for j in range(3)]
    ad2 = [bc(20 + j) for j in range(3)]

    @pl.loop(0, NPT // L)
    def _(j):
        sl16 = pl.ds(j * L, L)
        v0, v1, v2 = o0[sl16], o1[sl16], o2[sl16]
        h = [v0 * w2[0][jj] + v1 * w2[1][jj] + v2 * w2[2][jj] for jj in range(3)]
        rng[0, sl16] = h[0] * ad2[0] + h[1] * ad2[1] + h[2] * ad2[2]
        rng[1, sl16] = h[0]
        rng[2, sl16] = h[1]
        rng[3, sl16] = h[2]

    for k in range(4):
        pltpu.sync_copy(rng.at[k], sh_tbl.at[pl.ds(k * NP + nbase, NPT)])
    plsc.subcore_barrier()
    for k, tb in enumerate([tb0, tb1, tb2, tb3]):
        pltpu.sync_copy(sh_tbl.at[pl.ds(k * NP, NP)], tb)

    zero_accs([a0, a1, a2, a3])

    @pl.loop(0, EPT // L, unroll=4)
    def _(i):
        s16 = srcv[pl.ds(i * L, L)]
        d16 = dstv[pl.ds(i * L, L)]
        ed = plsc.load_gather(tb0, [d16])
        h0 = plsc.load_gather(tb1, [s16])
        h1 = plsc.load_gather(tb2, [s16])
        h2 = plsc.load_gather(tb3, [s16])
        es = h0 * as2[0] + h1 * as2[1] + h2 * as2[2]
        w = jnp.exp(_leaky(es + ed))
        plsc.addupdate_scatter(a0, [d16], w)
        plsc.addupdate_scatter(a1, [d16], w * h0)
        plsc.addupdate_scatter(a2, [d16], w * h1)
        plsc.addupdate_scatter(a3, [d16], w * h2)

    merge_accs([a0, a1, a2, a3])

    b20, b21, b22 = bc(23), bc(24), bc(25)

    @pl.loop(0, NPT // L)
    def _(j):
        sl16 = pl.ds(j * L, L)
        own = pl.ds(nbase + j * L, L)
        ed = tb0[own]
        h0, h1, h2 = tb1[own], tb2[own], tb3[own]
        es = h0 * as2[0] + h1 * as2[1] + h2 * as2[2]
        wv = jnp.exp(_leaky(es + ed))
        den = rng[0, sl16] + wv
        o0[sl16] = jnp.maximum((rng[1, sl16] + wv * h0) / den + b20, 0.0)
        o1[sl16] = jnp.maximum((rng[2, sl16] + wv * h1) / den + b21, 0.0)
        o2[sl16] = jnp.maximum((rng[3, sl16] + wv * h2) / den + b22, 0.0)

    w30, w31, w32 = bc(26), bc(27), bc(28)

    @pl.loop(0, NPT // L)
    def _(j):
        sl16 = pl.ds(j * L, L)
        rng[0, sl16] = o0[sl16] * w30 + o1[sl16] * w31 + o2[sl16] * w32

    pltpu.sync_copy(rng.at[0], sh_tbl.at[pl.ds(nbase, NPT)])
    plsc.subcore_barrier()
    pltpu.sync_copy(sh_tbl.at[pl.ds(0, NP)], tb0)

    as3, ad3, b3 = bc(29), bc(30), bc(31)
    zero_accs([a0, a1])

    @pl.loop(0, EPT // L, unroll=4)
    def _(i):
        s16 = srcv[pl.ds(i * L, L)]
        d16 = dstv[pl.ds(i * L, L)]
        hs = plsc.load_gather(tb0, [s16])
        hd = plsc.load_gather(tb0, [d16])
        w = jnp.exp(_leaky(as3 * hs + ad3 * hd))
        plsc.addupdate_scatter(a0, [d16], w)
        plsc.addupdate_scatter(a1, [d16], w * hs)

    merge_accs([a0, a1])

    iota = lax.iota(jnp.int32, L)

    @pl.loop(0, NPT // L, init_carry=zeros)
    def partial(j, acc):
        sl16 = pl.ds(j * L, L)
        hv = tb0[pl.ds(nbase + j * L, L)]
        wv = jnp.exp(_leaky(as3 * hv + ad3 * hv))
        den = rng[0, sl16] + wv
        o3 = (rng[1, sl16] + wv * hv) / den + b3
        ids = nbase + j * L + iota
        t = jnp.where(ids < N, jnp.exp(o3), 0.0)
        obuf[sl16] = t
        return acc + t

    pv[...] = partial
    pltpu.sync_copy(pv, sh_part.at[pl.ds(tid * L, L)])
    plsc.subcore_barrier()
    pltpu.sync_copy(sh_part, pbuf)

    tot = zeros
    for sl in range(NT):
        tot = tot + pbuf[pl.ds(sl * L, L)]
    totv = lax.broadcast_in_dim(jnp.sum(tot), (L,), ())
    inv = jnp.full((L,), 1.0, jnp.float32) / totv

    @pl.loop(0, NPT // L)
    def _(j):
        sl16 = pl.ds(j * L, L)
        obuf[sl16] = obuf[sl16] * inv

    pltpu.sync_copy(obuf, out_h.at[pl.ds(nbase, NPT)])


def _gat_sc(src, dst, xp, consts):
    mesh = plsc.VectorSubcoreMesh(core_axis_name="c", subcore_axis_name="s",
                                  num_cores=1)
    f = pl.kernel(
        _body,
        out_type=jax.ShapeDtypeStruct((NP,), jnp.float32),
        mesh=mesh,
        compiler_params=pltpu.CompilerParams(needs_layout_passes=False),
        scratch_types=[
            pltpu.VMEM((EPT,), jnp.int32),
            pltpu.VMEM((EPT,), jnp.int32),
            pltpu.VMEM((NP,), jnp.float32),
            pltpu.VMEM((NP,), jnp.float32),
            pltpu.VMEM((NP,), jnp.float32),
            pltpu.VMEM((NP,), jnp.float32),
            pltpu.VMEM((NP,), jnp.float32),
            pltpu.VMEM((NP,), jnp.float32),
            pltpu.VMEM((NP,), jnp.float32),
            pltpu.VMEM((NP,), jnp.float32),
            pltpu.VMEM((NPT,), jnp.float32),
            pltpu.VMEM((NPT,), jnp.float32),
            pltpu.VMEM((NPT,), jnp.float32),
            pltpu.VMEM((8, NPT), jnp.float32),
            pltpu.VMEM((NPT,), jnp.float32),
            pltpu.VMEM((NPT,), jnp.float32),
            pltpu.VMEM((NP,), jnp.int32),
            pltpu.VMEM((L,), jnp.float32),
            pltpu.VMEM((NT * L,), jnp.float32),
            pltpu.VMEM((32,), jnp.float32),
            pltpu.VMEM_SHARED((NP,), jnp.float32),
            pltpu.VMEM_SHARED((NP,), jnp.float32),
            pltpu.VMEM_SHARED((NP,), jnp.float32),
            pltpu.VMEM_SHARED((NP,), jnp.float32),
            pltpu.VMEM_SHARED((4 * NP,), jnp.float32),
            pltpu.VMEM_SHARED((NT * L,), jnp.float32),
            pltpu.SemaphoreType.DMA,
        ],
    )
    return f(src, dst, xp, consts)


def kernel(x, edge_index, W1, att_src1, att_dst1, b1, W2, att_src2, att_dst2,
           b2, W3, att_src3, att_dst3, b3, phi1, phi2):
    xs = x[:, 0]
    xp = jnp.concatenate([xs, jnp.zeros((NP - N,), jnp.float32)])
    src = edge_index[0]
    dst = edge_index[1]
    consts = jnp.concatenate([
        (W1[0] @ att_src1)[None], (W1[0] @ att_dst1)[None],
        W1[0], b1,
        W2.reshape(-1),
        att_src2, att_dst2, b2,
        W3[:, 0], att_src3, att_dst3, b3,
    ]).astype(jnp.float32)
    out_pad = _gat_sc(src, dst, xp, consts)
    return out_pad[:N, None]


# speedup vs baseline: 1.3207x; 1.3207x over previous
import jax
import jax.numpy as jnp
from jax import lax
from jax.experimental import pallas as pl
from jax.experimental.pallas import tpu as pltpu
from jax.experimental.pallas import tpu_sc as plsc

N = 10000
NP = 10240
E = 160000
NT = 16
EPT = E // NT
NPT = NP // NT
L = 16


def _leaky(e):
    return jnp.maximum(e, 0.0) + 0.2 * jnp.minimum(e, 0.0)


def _body(src_h, dst_h, x_h, c_h,
          out_h,
          srcv, dstv, tb0, tb1, tb2, tb3,
          a0, a1, a2, a3, o0, o1, o2, rng, obuf, zbuf, idxv, pv, pbuf, cv,
          sh_a0, sh_a1, sh_a2, sh_a3, sh_tbl, sh_part, sem):
    tid = lax.axis_index("s")
    ebase = tid * EPT
    nbase = tid * NPT
    zeros = jnp.zeros((L,), jnp.float32)

    def bc(k):
        return plsc.load_gather(cv, [jnp.full((L,), k, jnp.int32)])

    pltpu.sync_copy(src_h.at[pl.ds(ebase, EPT)], srcv)
    pltpu.sync_copy(dst_h.at[pl.ds(ebase, EPT)], dstv)
    pltpu.sync_copy(x_h, tb2)
    pltpu.sync_copy(c_h, cv)

    idc = lax.iota(jnp.int32, L)

    @pl.loop(0, NPT // L)
    def _(j):
        zbuf[pl.ds(j * L, L)] = zeros

    @pl.loop(0, NP // L)
    def _(j):
        idxv[pl.ds(j * L, L)] = j * L + idc

    sh_accs = [sh_a0, sh_a1, sh_a2, sh_a3]

    def zero_accs(refs):
        @pl.loop(0, NP // L)
        def _(j):
            for r in refs:
                r[pl.ds(j * L, L)] = zeros
        for k in range(len(refs)):
            pltpu.sync_copy(zbuf, sh_accs[k].at[pl.ds(nbase, NPT)])

    def merge_accs(refs):
        plsc.subcore_barrier()
        for k, r in enumerate(refs):
            pltpu.sync_copy(r, sh_accs[k].at[idxv], add=True)
        plsc.subcore_barrier()
        for k in range(len(refs)):
            pltpu.sync_copy(sh_accs[k].at[pl.ds(nbase, NPT)], rng.at[k])

    cs1 = bc(0)
    cd1 = bc(1)
    zero_accs([a0, a1])

    G = 4

    @pl.loop(0, EPT // (G * L))
    def _(i):
        s = [srcv[pl.ds((i * G + g) * L, L)] for g in range(G)]
        d = [dstv[pl.ds((i * G + g) * L, L)] for g in range(G)]
        xs = [plsc.load_gather(tb2, [s[g]]) for g in range(G)]
        xd = [plsc.load_gather(tb2, [d[g]]) for g in range(G)]
        w = [jnp.exp(_leaky(cs1 * xs[g] + cd1 * xd[g])) for g in range(G)]
        for g in range(G):
            plsc.addupdate_scatter(a0, [d[g]], w[g])
            plsc.addupdate_scatter(a1, [d[g]], w[g] * xs[g])

    merge_accs([a0, a1])

    w10, w11, w12 = bc(2), bc(3), bc(4)
    b10, b11, b12 = bc(5), bc(6), bc(7)

    @pl.loop(0, NPT // L)
    def _(j):
        sl16 = pl.ds(j * L, L)
        xv = tb2[pl.ds(nbase + j * L, L)]
        wv = jnp.exp(_leaky(cs1 * xv + cd1 * xv))
        den = rng[0, sl16] + wv
        s = rng[1, sl16] + wv * xv
        r = s / den
        o0[sl16] = jnp.maximum(w10 * r + b10, 0.0)
        o1[sl16] = jnp.maximum(w11 * r + b11, 0.0)
        o2[sl16] = jnp.maximum(w12 * r + b12, 0.0)

    w2 = [[bc(8 + 3 * i + j) for j in range(3)] for i in range(3)]
    as2 = [bc(17 + j) for j in range(3)]
    ad2 = [bc(20 + j) for j in range(3)]

    @pl.loop(0, NPT // L)
    def _(j):
        sl16 = pl.ds(j * L, L)
        v0, v1, v2 = o0[sl16], o1[sl16], o2[sl16]
        h = [v0 * w2[0][jj] + v1 * w2[1][jj] + v2 * w2[2][jj] for jj in range(3)]
        rng[0, sl16] = h[0] * ad2[0] + h[1] * ad2[1] + h[2] * ad2[2]
        rng[1, sl16] = h[0]
        rng[2, sl16] = h[1]
        rng[3, sl16] = h[2]

    for k in range(4):
        pltpu.sync_copy(rng.at[k], sh_tbl.at[pl.ds(k * NP + nbase, NPT)])
    plsc.subcore_barrier()
    for k, tb in enumerate([tb0, tb1, tb2, tb3]):
        pltpu.sync_copy(sh_tbl.at[pl.ds(k * NP, NP)], tb)

    zero_accs([a0, a1, a2, a3])

    @pl.loop(0, EPT // (G * L))
    def _(i):
        s = [srcv[pl.ds((i * G + g) * L, L)] for g in range(G)]
        d = [dstv[pl.ds((i * G + g) * L, L)] for g in range(G)]
        ed = [plsc.load_gather(tb0, [d[g]]) for g in range(G)]
        h0 = [plsc.load_gather(tb1, [s[g]]) for g in range(G)]
        h1 = [plsc.load_gather(tb2, [s[g]]) for g in range(G)]
        h2 = [plsc.load_gather(tb3, [s[g]]) for g in range(G)]
        es = [h0[g] * as2[0] + h1[g] * as2[1] + h2[g] * as2[2]
              for g in range(G)]
        w = [jnp.exp(_leaky(es[g] + ed[g])) for g in range(G)]
        for g in range(G):
            plsc.addupdate_scatter(a0, [d[g]], w[g])
            plsc.addupdate_scatter(a1, [d[g]], w[g] * h0[g])
            plsc.addupdate_scatter(a2, [d[g]], w[g] * h1[g])
            plsc.addupdate_scatter(a3, [d[g]], w[g] * h2[g])

    merge_accs([a0, a1, a2, a3])

    b20, b21, b22 = bc(23), bc(24), bc(25)

    @pl.loop(0, NPT // L)
    def _(j):
        sl16 = pl.ds(j * L, L)
        own = pl.ds(nbase + j * L, L)
        ed = tb0[own]
        h0, h1, h2 = tb1[own], tb2[own], tb3[own]
        es = h0 * as2[0] + h1 * as2[1] + h2 * as2[2]
        wv = jnp.exp(_leaky(es + ed))
        den = rng[0, sl16] + wv
        o0[sl16] = jnp.maximum((rng[1, sl16] + wv * h0) / den + b20, 0.0)
        o1[sl16] = jnp.maximum((rng[2, sl16] + wv * h1) / den + b21, 0.0)
        o2[sl16] = jnp.maximum((rng[3, sl16] + wv * h2) / den + b22, 0.0)

    w30, w31, w32 = bc(26), bc(27), bc(28)

    @pl.loop(0, NPT // L)
    def _(j):
        sl16 = pl.ds(j * L, L)
        rng[0, sl16] = o0[sl16] * w30 + o1[sl16] * w31 + o2[sl16] * w32

    pltpu.sync_copy(rng.at[0], sh_tbl.at[pl.ds(nbase, NPT)])
    plsc.subcore_barrier()
    pltpu.sync_copy(sh_tbl.at[pl.ds(0, NP)], tb0)

    as3, ad3, b3 = bc(29), bc(30), bc(31)
    zero_accs([a0, a1])

    @pl.loop(0, EPT // (G * L))
    def _(i):
        s = [srcv[pl.ds((i * G + g) * L, L)] for g in range(G)]
        d = [dstv[pl.ds((i * G + g) * L, L)] for g in range(G)]
        hs = [plsc.load_gather(tb0, [s[g]]) for g in range(G)]
        hd = [plsc.load_gather(tb0, [d[g]]) for g in range(G)]
        w = [jnp.exp(_leaky(as3 * hs[g] + ad3 * hd[g])) for g in range(G)]
        for g in range(G):
            plsc.addupdate_scatter(a0, [d[g]], w[g])
            plsc.addupdate_scatter(a1, [d[g]], w[g] * hs[g])

    merge_accs([a0, a1])

    iota = lax.iota(jnp.int32, L)

    @pl.loop(0, NPT // L, init_carry=zeros)
    def partial(j, acc):
        sl16 = pl.ds(j * L, L)
        hv = tb0[pl.ds(nbase + j * L, L)]
        wv = jnp.exp(_leaky(as3 * hv + ad3 * hv))
        den = rng[0, sl16] + wv
        o3 = (rng[1, sl16] + wv * hv) / den + b3
        ids = nbase + j * L + iota
        t = jnp.where(ids < N, jnp.exp(o3), 0.0)
        obuf[sl16] = t
        return acc + t

    pv[...] = partial
    pltpu.sync_copy(pv, sh_part.at[pl.ds(tid * L, L)])
    plsc.subcore_barrier()
    pltpu.sync_copy(sh_part, pbuf)

    tot = zeros
    for sl in range(NT):
        tot = tot + pbuf[pl.ds(sl * L, L)]
    totv = lax.broadcast_in_dim(jnp.sum(tot), (L,), ())
    inv = jnp.full((L,), 1.0, jnp.float32) / totv

    @pl.loop(0, NPT // L)
    def _(j):
        sl16 = pl.ds(j * L, L)
        obuf[sl16] = obuf[sl16] * inv

    pltpu.sync_copy(obuf, out_h.at[pl.ds(nbase, NPT)])


def _gat_sc(src, dst, xp, consts):
    mesh = plsc.VectorSubcoreMesh(core_axis_name="c", subcore_axis_name="s",
                                  num_cores=1)
    f = pl.kernel(
        _body,
        out_type=jax.ShapeDtypeStruct((NP,), jnp.float32),
        mesh=mesh,
        compiler_params=pltpu.CompilerParams(needs_layout_passes=False),
        scratch_types=[
            pltpu.VMEM((EPT,), jnp.int32),
            pltpu.VMEM((EPT,), jnp.int32),
            pltpu.VMEM((NP,), jnp.float32),
            pltpu.VMEM((NP,), jnp.float32),
            pltpu.VMEM((NP,), jnp.float32),
            pltpu.VMEM((NP,), jnp.float32),
            pltpu.VMEM((NP,), jnp.float32),
            pltpu.VMEM((NP,), jnp.float32),
            pltpu.VMEM((NP,), jnp.float32),
            pltpu.VMEM((NP,), jnp.float32),
            pltpu.VMEM((NPT,), jnp.float32),
            pltpu.VMEM((NPT,), jnp.float32),
            pltpu.VMEM((NPT,), jnp.float32),
            pltpu.VMEM((8, NPT), jnp.float32),
            pltpu.VMEM((NPT,), jnp.float32),
            pltpu.VMEM((NPT,), jnp.float32),
            pltpu.VMEM((NP,), jnp.int32),
            pltpu.VMEM((L,), jnp.float32),
            pltpu.VMEM((NT * L,), jnp.float32),
            pltpu.VMEM((32,), jnp.float32),
            pltpu.VMEM_SHARED((NP,), jnp.float32),
            pltpu.VMEM_SHARED((NP,), jnp.float32),
            pltpu.VMEM_SHARED((NP,), jnp.float32),
            pltpu.VMEM_SHARED((NP,), jnp.float32),
            pltpu.VMEM_SHARED((4 * NP,), jnp.float32),
            pltpu.VMEM_SHARED((NT * L,), jnp.float32),
            pltpu.SemaphoreType.DMA,
        ],
    )
    return f(src, dst, xp, consts)


def kernel(x, edge_index, W1, att_src1, att_dst1, b1, W2, att_src2, att_dst2,
           b2, W3, att_src3, att_dst3, b3, phi1, phi2):
    xs = x[:, 0]
    xp = jnp.concatenate([xs, jnp.zeros((NP - N,), jnp.float32)])
    src = edge_index[0]
    dst = edge_index[1]
    consts = jnp.concatenate([
        (W1[0] @ att_src1)[None], (W1[0] @ att_dst1)[None],
        W1[0], b1,
        W2.reshape(-1),
        att_src2, att_dst2, b2,
        W3[:, 0], att_src3, att_dst3, b3,
    ]).astype(jnp.float32)
    out_pad = _gat_sc(src, dst, xp, consts)
    return out_pad[:N, None]


# speedup vs baseline: 1.3673x; 1.0352x over previous
import jax
import jax.numpy as jnp
from jax import lax
from jax.experimental import pallas as pl
from jax.experimental.pallas import tpu as pltpu
from jax.experimental.pallas import tpu_sc as plsc

N = 10000
NP = 10240
E = 160000
NT = 16
EPT = E // NT
NPT = NP // NT
L = 16


def _leaky(e):
    return jnp.maximum(e, 0.0) + 0.2 * jnp.minimum(e, 0.0)


def _body(src_h, dst_h, x_h, c_h,
          out_h,
          srcv, dstv, tb0, tb1, tb2, tb3,
          a0, a1, a2, a3, o0, o1, o2, rng, obuf, zbuf, idxv, pv, pbuf, cv,
          sh_a0, sh_a1, sh_a2, sh_a3, sh_tbl, sh_part, sem):
    tid = lax.axis_index("s")
    ebase = tid * EPT
    nbase = tid * NPT
    zeros = jnp.zeros((L,), jnp.float32)

    def bc(k):
        return plsc.load_gather(cv, [jnp.full((L,), k, jnp.int32)])

    pltpu.sync_copy(src_h.at[pl.ds(ebase, EPT)], srcv)
    pltpu.sync_copy(dst_h.at[pl.ds(ebase, EPT)], dstv)
    pltpu.sync_copy(x_h, tb2)
    pltpu.sync_copy(c_h, cv)

    idc = lax.iota(jnp.int32, L)

    @pl.loop(0, NPT // L)
    def _(j):
        zbuf[pl.ds(j * L, L)] = zeros

    @pl.loop(0, NP // L)
    def _(j):
        idxv[pl.ds(j * L, L)] = j * L + idc

    sh_accs = [sh_a0, sh_a1, sh_a2, sh_a3]

    def zero_accs(refs):
        @pl.loop(0, NP // L)
        def _(j):
            for r in refs:
                r[pl.ds(j * L, L)] = zeros
        for k in range(len(refs)):
            pltpu.sync_copy(zbuf, sh_accs[k].at[pl.ds(nbase, NPT)])

    def merge_accs(refs):
        plsc.subcore_barrier()
        for k, r in enumerate(refs):
            pltpu.sync_copy(r, sh_accs[k].at[idxv], add=True)
        plsc.subcore_barrier()
        for k in range(len(refs)):
            pltpu.sync_copy(sh_accs[k].at[pl.ds(nbase, NPT)], rng.at[k])

    cs1 = bc(0)
    cd1 = bc(1)
    zero_accs([a0, a1])

    G = 8

    @pl.loop(0, EPT // (G * L))
    def _(i):
        s = [srcv[pl.ds((i * G + g) * L, L)] for g in range(G)]
        d = [dstv[pl.ds((i * G + g) * L, L)] for g in range(G)]
        xs = [plsc.load_gather(tb2, [s[g]]) for g in range(G)]
        xd = [plsc.load_gather(tb2, [d[g]]) for g in range(G)]
        w = [jnp.exp(_leaky(cs1 * xs[g] + cd1 * xd[g])) for g in range(G)]
        for g in range(G):
            plsc.addupdate_scatter(a0, [d[g]], w[g])
            plsc.addupdate_scatter(a1, [d[g]], w[g] * xs[g])

    merge_accs([a0, a1])

    w10, w11, w12 = bc(2), bc(3), bc(4)
    b10, b11, b12 = bc(5), bc(6), bc(7)

    @pl.loop(0, NPT // L)
    def _(j):
        sl16 = pl.ds(j * L, L)
        xv = tb2[pl.ds(nbase + j * L, L)]
        wv = jnp.exp(_leaky(cs1 * xv + cd1 * xv))
        den = rng[0, sl16] + wv
        s = rng[1, sl16] + wv * xv
        r = s / den
        o0[sl16] = jnp.maximum(w10 * r + b10, 0.0)
        o1[sl16] = jnp.maximum(w11 * r + b11, 0.0)
        o2[sl16] = jnp.maximum(w12 * r + b12, 0.0)

    w2 = [[bc(8 + 3 * i + j) for j in range(3)] for i in range(3)]
    as2 = [bc(17 + j) for j in range(3)]
    ad2 = [bc(20 + j) for j in range(3)]

    @pl.loop(0, NPT // L)
    def _(j):
        sl16 = pl.ds(j * L, L)
        v0, v1, v2 = o0[sl16], o1[sl16], o2[sl16]
        h = [v0 * w2[0][jj] + v1 * w2[1][jj] + v2 * w2[2][jj] for jj in range(3)]
        rng[0, sl16] = h[0] * ad2[0] + h[1] * ad2[1] + h[2] * ad2[2]
        rng[1, sl16] = h[0]
        rng[2, sl16] = h[1]
        rng[3, sl16] = h[2]

    for k in range(4):
        pltpu.sync_copy(rng.at[k], sh_tbl.at[pl.ds(k * NP + nbase, NPT)])
    plsc.subcore_barrier()
    for k, tb in enumerate([tb0, tb1, tb2, tb3]):
        pltpu.sync_copy(sh_tbl.at[pl.ds(k * NP, NP)], tb)

    zero_accs([a0, a1, a2, a3])

    @pl.loop(0, EPT // (G * L))
    def _(i):
        s = [srcv[pl.ds((i * G + g) * L, L)] for g in range(G)]
        d = [dstv[pl.ds((i * G + g) * L, L)] for g in range(G)]
        ed = [plsc.load_gather(tb0, [d[g]]) for g in range(G)]
        h0 = [plsc.load_gather(tb1, [s[g]]) for g in range(G)]
        h1 = [plsc.load_gather(tb2, [s[g]]) for g in range(G)]
        h2 = [plsc.load_gather(tb3, [s[g]]) for g in range(G)]
        es = [h0[g] * as2[0] + h1[g] * as2[1] + h2[g] * as2[2]
              for g in range(G)]
        w = [jnp.exp(_leaky(es[g] + ed[g])) for g in range(G)]
        for g in range(G):
            plsc.addupdate_scatter(a0, [d[g]], w[g])
            plsc.addupdate_scatter(a1, [d[g]], w[g] * h0[g])
            plsc.addupdate_scatter(a2, [d[g]], w[g] * h1[g])
            plsc.addupdate_scatter(a3, [d[g]], w[g] * h2[g])

    merge_accs([a0, a1, a2, a3])

    b20, b21, b22 = bc(23), bc(24), bc(25)

    @pl.loop(0, NPT // L)
    def _(j):
        sl16 = pl.ds(j * L, L)
        own = pl.ds(nbase + j * L, L)
        ed = tb0[own]
        h0, h1, h2 = tb1[own], tb2[own], tb3[own]
        es = h0 * as2[0] + h1 * as2[1] + h2 * as2[2]
        wv = jnp.exp(_leaky(es + ed))
        den = rng[0, sl16] + wv
        o0[sl16] = jnp.maximum((rng[1, sl16] + wv * h0) / den + b20, 0.0)
        o1[sl16] = jnp.maximum((rng[2, sl16] + wv * h1) / den + b21, 0.0)
        o2[sl16] = jnp.maximum((rng[3, sl16] + wv * h2) / den + b22, 0.0)

    w30, w31, w32 = bc(26), bc(27), bc(28)

    @pl.loop(0, NPT // L)
    def _(j):
        sl16 = pl.ds(j * L, L)
        rng[0, sl16] = o0[sl16] * w30 + o1[sl16] * w31 + o2[sl16] * w32

    pltpu.sync_copy(rng.at[0], sh_tbl.at[pl.ds(nbase, NPT)])
    plsc.subcore_barrier()
    pltpu.sync_copy(sh_tbl.at[pl.ds(0, NP)], tb0)

    as3, ad3, b3 = bc(29), bc(30), bc(31)
    zero_accs([a0, a1])

    @pl.loop(0, EPT // (G * L))
    def _(i):
        s = [srcv[pl.ds((i * G + g) * L, L)] for g in range(G)]
        d = [dstv[pl.ds((i * G + g) * L, L)] for g in range(G)]
        hs = [plsc.load_gather(tb0, [s[g]]) for g in range(G)]
        hd = [plsc.load_gather(tb0, [d[g]]) for g in range(G)]
        w = [jnp.exp(_leaky(as3 * hs[g] + ad3 * hd[g])) for g in range(G)]
        for g in range(G):
            plsc.addupdate_scatter(a0, [d[g]], w[g])
            plsc.addupdate_scatter(a1, [d[g]], w[g] * hs[g])

    merge_accs([a0, a1])

    iota = lax.iota(jnp.int32, L)

    @pl.loop(0, NPT // L, init_carry=zeros)
    def partial(j, acc):
        sl16 = pl.ds(j * L, L)
        hv = tb0[pl.ds(nbase + j * L, L)]
        wv = jnp.exp(_leaky(as3 * hv + ad3 * hv))
        den = rng[0, sl16] + wv
        o3 = (rng[1, sl16] + wv * hv) / den + b3
        ids = nbase + j * L + iota
        t = jnp.where(ids < N, jnp.exp(o3), 0.0)
        obuf[sl16] = t
        return acc + t

    pv[...] = partial
    pltpu.sync_copy(pv, sh_part.at[pl.ds(tid * L, L)])
    plsc.subcore_barrier()
    pltpu.sync_copy(sh_part, pbuf)

    tot = zeros
    for sl in range(NT):
        tot = tot + pbuf[pl.ds(sl * L, L)]
    totv = lax.broadcast_in_dim(jnp.sum(tot), (L,), ())
    inv = jnp.full((L,), 1.0, jnp.float32) / totv

    @pl.loop(0, NPT // L)
    def _(j):
        sl16 = pl.ds(j * L, L)
        obuf[sl16] = obuf[sl16] * inv

    pltpu.sync_copy(obuf, out_h.at[pl.ds(nbase, NPT)])


def _gat_sc(src, dst, xp, consts):
    mesh = plsc.VectorSubcoreMesh(core_axis_name="c", subcore_axis_name="s",
                                  num_cores=1)
    f = pl.kernel(
        _body,
        out_type=jax.ShapeDtypeStruct((NP,), jnp.float32),
        mesh=mesh,
        compiler_params=pltpu.CompilerParams(needs_layout_passes=False),
        scratch_types=[
            pltpu.VMEM((EPT,), jnp.int32),
            pltpu.VMEM((EPT,), jnp.int32),
            pltpu.VMEM((NP,), jnp.float32),
            pltpu.VMEM((NP,), jnp.float32),
            pltpu.VMEM((NP,), jnp.float32),
            pltpu.VMEM((NP,), jnp.float32),
            pltpu.VMEM((NP,), jnp.float32),
            pltpu.VMEM((NP,), jnp.float32),
            pltpu.VMEM((NP,), jnp.float32),
            pltpu.VMEM((NP,), jnp.float32),
            pltpu.VMEM((NPT,), jnp.float32),
            pltpu.VMEM((NPT,), jnp.float32),
            pltpu.VMEM((NPT,), jnp.float32),
            pltpu.VMEM((8, NPT), jnp.float32),
            pltpu.VMEM((NPT,), jnp.float32),
            pltpu.VMEM((NPT,), jnp.float32),
            pltpu.VMEM((NP,), jnp.int32),
            pltpu.VMEM((L,), jnp.float32),
            pltpu.VMEM((NT * L,), jnp.float32),
            pltpu.VMEM((32,), jnp.float32),
            pltpu.VMEM_SHARED((NP,), jnp.float32),
            pltpu.VMEM_SHARED((NP,), jnp.float32),
            pltpu.VMEM_SHARED((NP,), jnp.float32),
            pltpu.VMEM_SHARED((NP,), jnp.float32),
            pltpu.VMEM_SHARED((4 * NP,), jnp.float32),
            pltpu.VMEM_SHARED((NT * L,), jnp.float32),
            pltpu.SemaphoreType.DMA,
        ],
    )
    return f(src, dst, xp, consts)


def kernel(x, edge_index, W1, att_src1, att_dst1, b1, W2, att_src2, att_dst2,
           b2, W3, att_src3, att_dst3, b3, phi1, phi2):
    xs = x[:, 0]
    xp = jnp.concatenate([xs, jnp.zeros((NP - N,), jnp.float32)])
    src = edge_index[0]
    dst = edge_index[1]
    consts = jnp.concatenate([
        (W1[0] @ att_src1)[None], (W1[0] @ att_dst1)[None],
        W1[0], b1,
        W2.reshape(-1),
        att_src2, att_dst2, b2,
        W3[:, 0], att_src3, att_dst3, b3,
    ]).astype(jnp.float32)
    out_pad = _gat_sc(src, dst, xp, consts)
    return out_pad[:N, None]


# speedup vs baseline: 1.3768x; 1.0070x over previous
import jax
import jax.numpy as jnp
from jax import lax
from jax.experimental import pallas as pl
from jax.experimental.pallas import tpu as pltpu
from jax.experimental.pallas import tpu_sc as plsc

N = 10000
NP = 10240
E = 160000
NT = 16
EPT = E // NT
NPT = NP // NT
L = 16


def _leaky(e):
    return jnp.maximum(e, 0.0) + 0.2 * jnp.minimum(e, 0.0)


def _body(src_h, dst_h, x_h, c_h,
          out_h,
          srcv, dstv, tb0, tb1, tb2, tb3,
          a0, a1, a2, a3, o0, o1, o2, rng, obuf, zbuf, idxv, pv, pbuf, cv,
          sh_a0, sh_a1, sh_a2, sh_a3, sh_tbl, sh_part, sem):
    tid = lax.axis_index("s")
    ebase = tid * EPT
    nbase = tid * NPT
    zeros = jnp.zeros((L,), jnp.float32)

    def bc(k):
        return plsc.load_gather(cv, [jnp.full((L,), k, jnp.int32)])

    pltpu.sync_copy(src_h.at[pl.ds(ebase, EPT)], srcv)
    pltpu.sync_copy(dst_h.at[pl.ds(ebase, EPT)], dstv)
    pltpu.sync_copy(x_h, tb2)
    pltpu.sync_copy(c_h, cv)

    idc = lax.iota(jnp.int32, L)

    @pl.loop(0, NPT // L)
    def _(j):
        zbuf[pl.ds(j * L, L)] = zeros

    @pl.loop(0, NP // L)
    def _(j):
        idxv[pl.ds(j * L, L)] = j * L + idc

    sh_accs = [sh_a0, sh_a1, sh_a2, sh_a3]

    def zero_accs(refs):
        @pl.loop(0, NP // L)
        def _(j):
            for r in refs:
                r[pl.ds(j * L, L)] = zeros
        for k in range(len(refs)):
            pltpu.sync_copy(zbuf, sh_accs[k].at[pl.ds(nbase, NPT)])

    def merge_accs(refs):
        plsc.subcore_barrier()
        for k, r in enumerate(refs):
            pltpu.sync_copy(r, sh_accs[k].at[idxv], add=True)
        plsc.subcore_barrier()
        for k in range(len(refs)):
            pltpu.sync_copy(sh_accs[k].at[pl.ds(nbase, NPT)], rng.at[k])

    cs1 = bc(0)
    cd1 = bc(1)
    zero_accs([a0, a1])

    G = 16

    @pl.loop(0, EPT // (G * L))
    def _(i):
        s = [srcv[pl.ds((i * G + g) * L, L)] for g in range(G)]
        d = [dstv[pl.ds((i * G + g) * L, L)] for g in range(G)]
        xs = [plsc.load_gather(tb2, [s[g]]) for g in range(G)]
        xd = [plsc.load_gather(tb2, [d[g]]) for g in range(G)]
        w = [jnp.exp(_leaky(cs1 * xs[g] + cd1 * xd[g])) for g in range(G)]
        for g in range(G):
            plsc.addupdate_scatter(a0, [d[g]], w[g])
            plsc.addupdate_scatter(a1, [d[g]], w[g] * xs[g])

    merge_accs([a0, a1])

    w10, w11, w12 = bc(2), bc(3), bc(4)
    b10, b11, b12 = bc(5), bc(6), bc(7)

    @pl.loop(0, NPT // L)
    def _(j):
        sl16 = pl.ds(j * L, L)
        xv = tb2[pl.ds(nbase + j * L, L)]
        wv = jnp.exp(_leaky(cs1 * xv + cd1 * xv))
        den = rng[0, sl16] + wv
        s = rng[1, sl16] + wv * xv
        r = s / den
        o0[sl16] = jnp.maximum(w10 * r + b10, 0.0)
        o1[sl16] = jnp.maximum(w11 * r + b11, 0.0)
        o2[sl16] = jnp.maximum(w12 * r + b12, 0.0)

    w2 = [[bc(8 + 3 * i + j) for j in range(3)] for i in range(3)]
    as2 = [bc(17 + j) for j in range(3)]
    ad2 = [bc(20 + j) for j in range(3)]

    @pl.loop(0, NPT // L)
    def _(j):
        sl16 = pl.ds(j * L, L)
        v0, v1, v2 = o0[sl16], o1[sl16], o2[sl16]
        h = [v0 * w2[0][jj] + v1 * w2[1][jj] + v2 * w2[2][jj] for jj in range(3)]
        rng[0, sl16] = h[0] * ad2[0] + h[1] * ad2[1] + h[2] * ad2[2]
        rng[1, sl16] = h[0]
        rng[2, sl16] = h[1]
        rng[3, sl16] = h[2]

    for k in range(4):
        pltpu.sync_copy(rng.at[k], sh_tbl.at[pl.ds(k * NP + nbase, NPT)])
    plsc.subcore_barrier()
    for k, tb in enumerate([tb0, tb1, tb2, tb3]):
        pltpu.sync_copy(sh_tbl.at[pl.ds(k * NP, NP)], tb)

    zero_accs([a0, a1, a2, a3])

    @pl.loop(0, EPT // (G * L))
    def _(i):
        s = [srcv[pl.ds((i * G + g) * L, L)] for g in range(G)]
        d = [dstv[pl.ds((i * G + g) * L, L)] for g in range(G)]
        ed = [plsc.load_gather(tb0, [d[g]]) for g in range(G)]
        h0 = [plsc.load_gather(tb1, [s[g]]) for g in range(G)]
        h1 = [plsc.load_gather(tb2, [s[g]]) for g in range(G)]
        h2 = [plsc.load_gather(tb3, [s[g]]) for g in range(G)]
        es = [h0[g] * as2[0] + h1[g] * as2[1] + h2[g] * as2[2]
              for g in range(G)]
        w = [jnp.exp(_leaky(es[g] + ed[g])) for g in range(G)]
        for g in range(G):
            plsc.addupdate_scatter(a0, [d[g]], w[g])
            plsc.addupdate_scatter(a1, [d[g]], w[g] * h0[g])
            plsc.addupdate_scatter(a2, [d[g]], w[g] * h1[g])
            plsc.addupdate_scatter(a3, [d[g]], w[g] * h2[g])

    merge_accs([a0, a1, a2, a3])

    b20, b21, b22 = bc(23), bc(24), bc(25)

    @pl.loop(0, NPT // L)
    def _(j):
        sl16 = pl.ds(j * L, L)
        own = pl.ds(nbase + j * L, L)
        ed = tb0[own]
        h0, h1, h2 = tb1[own], tb2[own], tb3[own]
        es = h0 * as2[0] + h1 * as2[1] + h2 * as2[2]
        wv = jnp.exp(_leaky(es + ed))
        den = rng[0, sl16] + wv
        o0[sl16] = jnp.maximum((rng[1, sl16] + wv * h0) / den + b20, 0.0)
        o1[sl16] = jnp.maximum((rng[2, sl16] + wv * h1) / den + b21, 0.0)
        o2[sl16] = jnp.maximum((rng[3, sl16] + wv * h2) / den + b22, 0.0)

    w30, w31, w32 = bc(26), bc(27), bc(28)

    @pl.loop(0, NPT // L)
    def _(j):
        sl16 = pl.ds(j * L, L)
        rng[0, sl16] = o0[sl16] * w30 + o1[sl16] * w31 + o2[sl16] * w32

    pltpu.sync_copy(rng.at[0], sh_tbl.at[pl.ds(nbase, NPT)])
    plsc.subcore_barrier()
    pltpu.sync_copy(sh_tbl.at[pl.ds(0, NP)], tb0)

    as3, ad3, b3 = bc(29), bc(30), bc(31)
    zero_accs([a0, a1])

    @pl.loop(0, EPT // (G * L))
    def _(i):
        s = [srcv[pl.ds((i * G + g) * L, L)] for g in range(G)]
        d = [dstv[pl.ds((i * G + g) * L, L)] for g in range(G)]
        hs = [plsc.load_gather(tb0, [s[g]]) for g in range(G)]
        hd = [plsc.load_gather(tb0, [d[g]]) for g in range(G)]
        w = [jnp.exp(_leaky(as3 * hs[g] + ad3 * hd[g])) for g in range(G)]
        for g in range(G):
            plsc.addupdate_scatter(a0, [d[g]], w[g])
            plsc.addupdate_scatter(a1, [d[g]], w[g] * hs[g])

    merge_accs([a0, a1])

    iota = lax.iota(jnp.int32, L)

    @pl.loop(0, NPT // L, init_carry=zeros)
    def partial(j, acc):
        sl16 = pl.ds(j * L, L)
        hv = tb0[pl.ds(nbase + j * L, L)]
        wv = jnp.exp(_leaky(as3 * hv + ad3 * hv))
        den = rng[0, sl16] + wv
        o3 = (rng[1, sl16] + wv * hv) / den + b3
        ids = nbase + j * L + iota
        t = jnp.where(ids < N, jnp.exp(o3), 0.0)
        obuf[sl16] = t
        return acc + t

    pv[...] = partial
    pltpu.sync_copy(pv, sh_part.at[pl.ds(tid * L, L)])
    plsc.subcore_barrier()
    pltpu.sync_copy(sh_part, pbuf)

    tot = zeros
    for sl in range(NT):
        tot = tot + pbuf[pl.ds(sl * L, L)]
    totv = lax.broadcast_in_dim(jnp.sum(tot), (L,), ())
    inv = jnp.full((L,), 1.0, jnp.float32) / totv

    @pl.loop(0, NPT // L)
    def _(j):
        sl16 = pl.ds(j * L, L)
        obuf[sl16] = obuf[sl16] * inv

    pltpu.sync_copy(obuf, out_h.at[pl.ds(nbase, NPT)])


def _gat_sc(src, dst, xp, consts):
    mesh = plsc.VectorSubcoreMesh(core_axis_name="c", subcore_axis_name="s",
                                  num_cores=1)
    f = pl.kernel(
        _body,
        out_type=jax.ShapeDtypeStruct((NP,), jnp.float32),
        mesh=mesh,
        compiler_params=pltpu.CompilerParams(needs_layout_passes=False),
        scratch_types=[
            pltpu.VMEM((EPT,), jnp.int32),
            pltpu.VMEM((EPT,), jnp.int32),
            pltpu.VMEM((NP,), jnp.float32),
            pltpu.VMEM((NP,), jnp.float32),
            pltpu.VMEM((NP,), jnp.float32),
            pltpu.VMEM((NP,), jnp.float32),
            pltpu.VMEM((NP,), jnp.float32),
            pltpu.VMEM((NP,), jnp.float32),
            pltpu.VMEM((NP,), jnp.float32),
            pltpu.VMEM((NP,), jnp.float32),
            pltpu.VMEM((NPT,), jnp.float32),
            pltpu.VMEM((NPT,), jnp.float32),
            pltpu.VMEM((NPT,), jnp.float32),
            pltpu.VMEM((8, NPT), jnp.float32),
            pltpu.VMEM((NPT,), jnp.float32),
            pltpu.VMEM((NPT,), jnp.float32),
            pltpu.VMEM((NP,), jnp.int32),
            pltpu.VMEM((L,), jnp.float32),
            pltpu.VMEM((NT * L,), jnp.float32),
            pltpu.VMEM((32,), jnp.float32),
            pltpu.VMEM_SHARED((NP,), jnp.float32),
            pltpu.VMEM_SHARED((NP,), jnp.float32),
            pltpu.VMEM_SHARED((NP,), jnp.float32),
            pltpu.VMEM_SHARED((NP,), jnp.float32),
            pltpu.VMEM_SHARED((4 * NP,), jnp.float32),
            pltpu.VMEM_SHARED((NT * L,), jnp.float32),
            pltpu.SemaphoreType.DMA,
        ],
    )
    return f(src, dst, xp, consts)


def kernel(x, edge_index, W1, att_src1, att_dst1, b1, W2, att_src2, att_dst2,
           b2, W3, att_src3, att_dst3, b3, phi1, phi2):
    xs = x[:, 0]
    xp = jnp.concatenate([xs, jnp.zeros((NP - N,), jnp.float32)])
    src = edge_index[0]
    dst = edge_index[1]
    consts = jnp.concatenate([
        (W1[0] @ att_src1)[None], (W1[0] @ att_dst1)[None],
        W1[0], b1,
        W2.reshape(-1),
        att_src2, att_dst2, b2,
        W3[:, 0], att_src3, att_dst3, b3,
    ]).astype(jnp.float32)
    out_pad = _gat_sc(src, dst, xp, consts)
    return out_pad[:N, None]
